# Initial kernel scaffold; baseline (speedup 1.0000x reference)
import jax
import jax.numpy as jnp
from jax.experimental import pallas as pl


def kernel(node_features, neighbors, W, a, gamma, beta):
    raise NotImplementedError("write your pallas kernel here")


# speedup vs baseline: 7.3914x; 7.3914x over previous
import functools

import jax
import jax.numpy as jnp
from jax import lax
from jax.experimental import pallas as pl
from jax.experimental.pallas import tpu as pltpu
from jax.experimental.pallas import tpu_sc as plsc

N = 10000
K = 32
D = 128
H = 4
U = 32
HU = H * U

NW = 32
CH = 4
N_PAD = 10240
NODES_W = N_PAD // NW
CHUNKS_W = NODES_W // CH
NEG = -1000000000.0


def _splat(val, dtype=jnp.float32):
    return jnp.full((16,), val, dtype=dtype)


_GDN = lax.GatherDimensionNumbers(
    offset_dims=(), collapsed_slice_dims=(0,), start_index_map=(0,))


def _gather16(vec, idx16):
    return lax.gather(vec, idx16[:, None], _GDN, (1,),
                      mode=lax.GatherScatterMode.PROMISE_IN_BOUNDS)


def _last_lane(vec):
    return _gather16(vec, _splat(15, jnp.int32))


def _mm_body(x_ref, w_ref, asx_ref, anx_ref, h_ref, ss_ref, sn_ref):
    h = jnp.dot(x_ref[...], w_ref[...], preferred_element_type=jnp.float32)
    h_ref[...] = h
    ss_ref[...] = jnp.dot(h, asx_ref[...], preferred_element_type=jnp.float32)
    sn_ref[...] = jnp.dot(h, anx_ref[...], preferred_element_type=jnp.float32)


def _project(x_pad, W, a_self_m, a_nbr_m):
    blk = 1024
    grid = N_PAD // blk
    return pl.pallas_call(
        _mm_body,
        grid=(grid,),
        in_specs=[
            pl.BlockSpec((blk, D), lambda i: (i, 0)),
            pl.BlockSpec((D, HU), lambda i: (0, 0)),
            pl.BlockSpec((HU, H), lambda i: (0, 0)),
            pl.BlockSpec((HU, H), lambda i: (0, 0)),
        ],
        out_specs=[
            pl.BlockSpec((blk, HU), lambda i: (i, 0)),
            pl.BlockSpec((blk, H), lambda i: (i, 0)),
            pl.BlockSpec((blk, H), lambda i: (i, 0)),
        ],
        out_shape=[
            jax.ShapeDtypeStruct((N_PAD, HU), jnp.float32),
            jax.ShapeDtypeStruct((N_PAD, H), jnp.float32),
            jax.ShapeDtypeStruct((N_PAD, H), jnp.float32),
        ],
    )(x_pad, W, a_self_m, a_nbr_m)


def _attn_body(h_hbm, ss_hbm, sn_hbm, nbr_hbm, out_hbm,
               snbr_v, sself_v, adj_v, idx0_v, idx1_v, rows0_v, rows1_v,
               ctx_v, sem0, sem1):
    cid = lax.axis_index("c")
    sid = lax.axis_index("s")
    wid = cid * 16 + sid
    node0 = wid * NODES_W

    pltpu.sync_copy(sn_hbm, snbr_v)
    pltpu.sync_copy(ss_hbm.at[pl.ds(node0 * H, NODES_W * H)], sself_v)
    pltpu.sync_copy(nbr_hbm.at[pl.ds(node0, NODES_W)], adj_v)

    idx_bufs = (idx0_v, idx1_v)
    row_bufs = (rows0_v, rows1_v)
    sems = (sem0, sem1)

    def issue(g, slot):
        for j in range(CH):
            nl = jnp.minimum(g * CH + j, NODES_W - 1)
            for kc in range(2):
                nbr = adj_v[nl, pl.ds(kc * 16, 16)]
                idx = jnp.maximum(nbr - 1, 0)
                idx_bufs[slot][pl.ds(j * K + kc * 16, 16)] = idx
        pltpu.make_async_copy(
            h_hbm.at[idx_bufs[slot]], row_bufs[slot], sems[slot]).start()

    def compute(g, slot):
        rows = row_bufs[slot]
        for j in range(CH):
            nl = g * CH + j
            nbrs = []
            valids = []
            for kc in range(2):
                nbr = adj_v[nl, pl.ds(kc * 16, 16)]
                nbrs.append(jnp.maximum(nbr - 1, 0))
                valids.append(nbr > 0)
            evecs = []
            rinvs = []
            for h in range(H):
                hsplat = _splat(h, jnp.int32)
                scs = []
                sself = plsc.load_gather(
                    sself_v, [_splat(nl * H + h, jnp.int32)])
                for kc in range(2):
                    snbr = plsc.load_gather(snbr_v, [nbrs[kc] * H + h])
                    sc = sself + snbr
                    sc = jnp.where(sc > 0, sc, 0.2 * sc)
                    sc = jnp.where(valids[kc], sc, NEG)
                    scs.append(sc)
                mx = _last_lane(plsc.cummax(jnp.maximum(scs[0], scs[1])))
                e0 = jnp.exp(scs[0] - mx)
                e1 = jnp.exp(scs[1] - mx)
                den = _last_lane(plsc.cumsum(e0 + e1))
                evecs.append((e0, e1))
                rinvs.append(1.0 / den)

            def kbody(kc):
                def body(k, accs):
                    lane = _splat(k - kc * 16, jnp.int32)
                    row = j * K + k
                    out = list(accs)
                    for h in range(H):
                        eb = _gather16(evecs[h][kc], lane)
                        for uc in range(2):
                            c = h * 2 + uc
                            rv = rows[row, pl.ds(c * 16, 16)]
                            out[c] = out[c] + eb * rv
                    return tuple(out)
                return body

            accs = tuple(jnp.zeros((16,), jnp.float32) for _ in range(8))
            accs = lax.fori_loop(0, 16, kbody(0), accs)
            accs = lax.fori_loop(16, 32, kbody(1), accs)
            for c in range(8):
                ctx_v[j, pl.ds(c * 16, 16)] = accs[c] * rinvs[c // 2]
        pltpu.sync_copy(ctx_v, out_hbm.at[pl.ds(node0 + g * CH, CH)])

    issue(0, 0)

    def outer(t, carry):
        g0 = 2 * t
        issue(g0 + 1, 1)
        pltpu.make_async_copy(
            h_hbm.at[idx_bufs[0]], row_bufs[0], sems[0]).wait()
        compute(g0, 0)
        issue(jnp.minimum(g0 + 2, CHUNKS_W - 1), 0)
        pltpu.make_async_copy(
            h_hbm.at[idx_bufs[1]], row_bufs[1], sems[1]).wait()
        compute(g0 + 1, 1)
        return carry

    lax.fori_loop(0, CHUNKS_W // 2, outer, 0)
    pltpu.make_async_copy(h_hbm.at[idx_bufs[0]], row_bufs[0], sems[0]).wait()


def _attention(h, s_self, s_nbr, nbr_pad):
    mesh = plsc.VectorSubcoreMesh(core_axis_name="c", subcore_axis_name="s")
    kfn = pl.kernel(
        _attn_body,
        out_type=jax.ShapeDtypeStruct((N_PAD, HU), jnp.float32),
        mesh=mesh,
        scratch_types=[
            pltpu.VMEM((N_PAD * H,), jnp.float32),
            pltpu.VMEM((NODES_W * H,), jnp.float32),
            pltpu.VMEM((NODES_W, K), jnp.int32),
            pltpu.VMEM((CH * K,), jnp.int32),
            pltpu.VMEM((CH * K,), jnp.int32),
            pltpu.VMEM((CH * K, HU), jnp.float32),
            pltpu.VMEM((CH * K, HU), jnp.float32),
            pltpu.VMEM((CH, HU), jnp.float32),
            pltpu.SemaphoreType.DMA,
            pltpu.SemaphoreType.DMA,
        ],
        compiler_params=pltpu.CompilerParams(needs_layout_passes=False),
    )
    return kfn(h, s_self.reshape(-1), s_nbr.reshape(-1), nbr_pad)


def _ln_body(x_ref, g_ref, b_ref, o_ref):
    y = jnp.maximum(x_ref[...], 0.0)
    mean = jnp.mean(y, axis=-1, keepdims=True)
    var = jnp.mean((y - mean) ** 2, axis=-1, keepdims=True)
    o_ref[...] = (y - mean) / jnp.sqrt(var + 1e-3) * g_ref[...] + b_ref[...]


def _layernorm(ctx, gamma, beta):
    blk = 1024
    return pl.pallas_call(
        _ln_body,
        grid=(N_PAD // blk,),
        in_specs=[
            pl.BlockSpec((blk, HU), lambda i: (i, 0)),
            pl.BlockSpec((1, HU), lambda i: (0, 0)),
            pl.BlockSpec((1, HU), lambda i: (0, 0)),
        ],
        out_specs=pl.BlockSpec((blk, HU), lambda i: (i, 0)),
        out_shape=jax.ShapeDtypeStruct((N_PAD, HU), jnp.float32),
    )(ctx, gamma.reshape(1, HU), beta.reshape(1, HU))


@jax.jit
def kernel(node_features, neighbors, W, a, gamma, beta):
    x = node_features[0]
    x_pad = jnp.pad(x, ((0, N_PAD - N), (0, 0)))
    nbr_pad = jnp.pad(neighbors[0], ((0, N_PAD - N), (0, 0)))

    eye = jnp.eye(H, dtype=jnp.float32)
    a_self_m = (a[:, :U, None] * eye[:, None, :]).reshape(HU, H)
    a_nbr_m = (a[:, U:, None] * eye[:, None, :]).reshape(HU, H)

    h, s_self, s_nbr = _project(x_pad, W, a_self_m, a_nbr_m)
    ctx = _attention(h, s_self, s_nbr, nbr_pad)
    out = _layernorm(ctx, gamma, beta)
    return out[None, :N, :]
